# baseline (speedup 1.0000x reference)
import functools

import jax
import jax.numpy as jnp
from jax import lax
from jax.experimental import pallas as pl
from jax.experimental.pallas import tpu as pltpu
from jax.experimental.pallas import tpu_sc as plsc

N_NODES = 10000
D = 128
G_GRAPHS = 128
C_OUT = 10

N_SC = 10112
N_PAD = 10240
RBLK = 512
NRB = N_PAD // RBLK
QSUB = RBLK // 128

NC, NS = 2, 16
NW = NC * NS
CHUNK = 128
CPW = 79
E_PAD = NW * CPW * CHUNK
ROWS_PT = N_SC // NS
CPW0, CPW1 = 153, 5

_MESH = plsc.VectorSubcoreMesh(core_axis_name="c", subcore_axis_name="s")


def _sc_deg(dsts):
    @functools.partial(
        pl.kernel,
        out_type=jax.ShapeDtypeStruct((NC, NS, 1, 640), jnp.float32),
        mesh=_MESH,
        scratch_types=[
            pltpu.VMEM((CPW, CHUNK), jnp.int32),
            pltpu.VMEM((CHUNK,), jnp.float32),
            pltpu.VMEM((640,), jnp.float32),
            pltpu.VMEM_SHARED((N_SC + 16,), jnp.float32),
        ],
    )
    def k(dst_hbm, deg_hbm, idx_v, ones_v, z_v, deg_sh):
        c = lax.axis_index("c")
        s = lax.axis_index("s")
        w = c * NS + s
        pltpu.sync_copy(dst_hbm.at[w], idx_v)

        @pl.loop(0, CHUNK // 16)
        def _(i):
            ones_v[pl.ds(i * 16, 16)] = jnp.full((16,), 1.0, jnp.float32)

        @pl.loop(0, 640 // 16)
        def _(i):
            z_v[pl.ds(i * 16, 16)] = jnp.zeros((16,), jnp.float32)

        pltpu.sync_copy(z_v.at[pl.ds(0, ROWS_PT)],
                        deg_sh.at[pl.ds(s * ROWS_PT, ROWS_PT)])
        plsc.subcore_barrier()

        @pl.loop(0, CPW)
        def _(j):
            pltpu.sync_copy(ones_v, deg_sh.at[idx_v.at[j]], add=True)

        plsc.subcore_barrier()
        pltpu.sync_copy(deg_sh.at[pl.ds(s * ROWS_PT, 640)], z_v)
        pltpu.sync_copy(z_v, deg_hbm.at[c, s, 0, pl.ds(0, 640)])

    return k(dsts)


def _sc_spmm(g, idx0, idx1):
    @functools.partial(
        pl.kernel,
        out_type=jax.ShapeDtypeStruct((NC, N_PAD, D), jnp.float32),
        mesh=_MESH,
        scratch_types=[
            pltpu.VMEM((2, CHUNK), jnp.int32),
            pltpu.VMEM((2, CHUNK), jnp.int32),
            pltpu.VMEM((2, CHUNK), jnp.int32),
            pltpu.VMEM((2, CHUNK), jnp.int32),
            pltpu.VMEM((CHUNK, D), jnp.float32),
            pltpu.VMEM((CHUNK, D), jnp.float32),
            pltpu.VMEM_SHARED((N_SC, D), jnp.float32),
            pltpu.SemaphoreType.DMA,
            pltpu.SemaphoreType.DMA,
            pltpu.SemaphoreType.DMA,
            pltpu.SemaphoreType.DMA,
            pltpu.SemaphoreType.DMA,
            pltpu.SemaphoreType.DMA,
        ],
    )
    def k(g_hbm, idx0_hbm, idx1_hbm, out_hbm,
          ixa, ixb, ixc, ixd, bufa, bufb, acc_sh,
          sga, sgb, sia, sib, sic, sid):
        c = lax.axis_index("c")
        s = lax.axis_index("s")

        @pl.loop(0, CHUNK)
        def _(i):
            for t in range(D // 16):
                bufa[i, pl.ds(t * 16, 16)] = jnp.zeros((16,), jnp.float32)

        base = s * ROWS_PT
        for off, nrows in ((0, 128), (128, 128), (256, 128), (384, 128),
                           (512, 120)):
            pltpu.sync_copy(bufa.at[pl.ds(0, nrows)],
                            acc_sh.at[pl.ds(base + off, nrows)])
        plsc.subcore_barrier()

        def pipeline(idx_hbm, cpw):
            jb = s * cpw
            pltpu.sync_copy(idx_hbm.at[jb], ixa)
            pltpu.async_copy(g_hbm.at[ixa.at[0]], bufa, sga)
            pltpu.async_copy(idx_hbm.at[jb + 1], ixb, sib)
            pltpu.async_copy(idx_hbm.at[jb + 2], ixc, sic)

            @pl.loop(0, (cpw - 1) // 4)
            def _(i):
                j = i * 4
                pltpu.make_async_copy(idx_hbm.at[jb + j + 1], ixb, sib).wait()
                pltpu.async_copy(g_hbm.at[ixb.at[0]], bufb, sgb)
                pltpu.make_async_copy(g_hbm.at[ixa.at[0]], bufa, sga).wait()
                pltpu.sync_copy(bufa, acc_sh.at[ixa.at[1]], add=True)
                pltpu.async_copy(idx_hbm.at[jb + j + 3], ixd, sid)
                pltpu.make_async_copy(idx_hbm.at[jb + j + 2], ixc, sic).wait()
                pltpu.async_copy(g_hbm.at[ixc.at[0]], bufa, sga)
                pltpu.make_async_copy(g_hbm.at[ixb.at[0]], bufb, sgb).wait()
                pltpu.sync_copy(bufb, acc_sh.at[ixb.at[1]], add=True)
                pltpu.async_copy(idx_hbm.at[jb + j + 4], ixa, sia)
                pltpu.make_async_copy(idx_hbm.at[jb + j + 3], ixd, sid).wait()
                pltpu.async_copy(g_hbm.at[ixd.at[0]], bufb, sgb)
                pltpu.make_async_copy(g_hbm.at[ixc.at[0]], bufa, sga).wait()
                pltpu.sync_copy(bufa, acc_sh.at[ixc.at[1]], add=True)
                pltpu.async_copy(idx_hbm.at[jb + j + 5], ixb, sib)
                pltpu.make_async_copy(idx_hbm.at[jb + j + 4], ixa, sia).wait()
                pltpu.async_copy(g_hbm.at[ixa.at[0]], bufa, sga)
                pltpu.make_async_copy(g_hbm.at[ixd.at[0]], bufb, sgb).wait()
                pltpu.sync_copy(bufb, acc_sh.at[ixd.at[1]], add=True)
                pltpu.async_copy(idx_hbm.at[jb + j + 6], ixc, sic)

            pltpu.make_async_copy(g_hbm.at[ixa.at[0]], bufa, sga).wait()
            pltpu.sync_copy(bufa, acc_sh.at[ixa.at[1]], add=True)
            pltpu.make_async_copy(idx_hbm.at[jb + cpw], ixb, sib).wait()
            pltpu.make_async_copy(idx_hbm.at[jb + cpw + 1], ixc, sic).wait()

        @pl.when(c == 0)
        def _():
            pipeline(idx0_hbm, CPW0)

        @pl.when(c == 1)
        def _():
            pipeline(idx1_hbm, CPW1)

        plsc.subcore_barrier()
        for off, nrows in ((0, 128), (128, 128), (256, 128), (384, 128),
                           (512, 120)):
            pltpu.sync_copy(acc_sh.at[pl.ds(base + off, nrows)],
                            bufa.at[pl.ds(0, nrows)])
            pltpu.sync_copy(bufa.at[pl.ds(0, nrows)],
                            out_hbm.at[c, pl.ds(base + off, nrows)])

    return k(g, idx0, idx1)


def _dinv_block(deg_ref):
    ii = lax.broadcasted_iota(jnp.int32, (128, 128), 0)
    jj = lax.broadcasted_iota(jnp.int32, (128, 128), 1)
    ones = jnp.ones((128, 128), jnp.float32)
    pieces = []
    for q in range(QSUB):
        d = deg_ref[0, q] + deg_ref[1, q] + 1.0
        row = lax.rsqrt(d)
        diag = jnp.where(ii == jj, jnp.broadcast_to(row, (128, 128)), 0.0)
        pieces.append(jnp.dot(diag, ones,
                              preferred_element_type=jnp.float32))
    return jnp.concatenate(pieces, axis=0)


def _tc_g1(degp, x, w1):
    def body(deg_ref, x_ref, w_ref, g_ref):
        dinvb = _dinv_block(deg_ref)
        g_ref[...] = dinvb * jnp.dot(x_ref[...], w_ref[...],
                                     preferred_element_type=jnp.float32)

    return pl.pallas_call(
        body,
        grid=(NRB,),
        in_specs=[
            pl.BlockSpec((NC, QSUB, 1, 128), lambda r: (0, r, 0, 0)),
            pl.BlockSpec((RBLK, D), lambda r: (r, 0)),
            pl.BlockSpec((D, D), lambda r: (0, 0)),
        ],
        out_specs=pl.BlockSpec((RBLK, D), lambda r: (r, 0)),
        out_shape=jax.ShapeDtypeStruct((N_PAD, D), jnp.float32),
    )(degp, x, w1)


def _tc_layer(sp, gprev, degp, brow, w):
    def body(sp_ref, g_ref, deg_ref, b_ref, w_ref, out_ref):
        dinvb = _dinv_block(deg_ref)
        ssum = sp_ref[0] + sp_ref[1]
        a = jnp.maximum(dinvb * (ssum + g_ref[...]) + b_ref[...], 0.0)
        out_ref[...] = dinvb * jnp.dot(a, w_ref[...],
                                       preferred_element_type=jnp.float32)

    return pl.pallas_call(
        body,
        grid=(NRB,),
        in_specs=[
            pl.BlockSpec((NC, RBLK, D), lambda r: (0, r, 0)),
            pl.BlockSpec((RBLK, D), lambda r: (r, 0)),
            pl.BlockSpec((NC, QSUB, 1, 128), lambda r: (0, r, 0, 0)),
            pl.BlockSpec((1, D), lambda r: (0, 0)),
            pl.BlockSpec((D, D), lambda r: (0, 0)),
        ],
        out_specs=pl.BlockSpec((RBLK, D), lambda r: (r, 0)),
        out_shape=jax.ShapeDtypeStruct((N_PAD, D), jnp.float32),
    )(sp, gprev, degp, brow, w)


def _tc_final(sp, gprev, degp, brow, batchp, wlp, blp):
    def body(sp_ref, g_ref, deg_ref, b_ref, bt_ref, wl_ref, bl_ref, out_ref,
             pooled, counts):
        r = pl.program_id(0)

        @pl.when(r == 0)
        def _():
            pooled[...] = jnp.zeros_like(pooled)
            counts[...] = jnp.zeros_like(counts)

        dinvb = _dinv_block(deg_ref)
        h = dinvb * (sp_ref[0] + sp_ref[1] + g_ref[...]) + b_ref[...]
        rid = r * RBLK + lax.broadcasted_iota(jnp.int32, (RBLK, 1), 0)
        h = jnp.where(rid < N_NODES, h, 0.0)
        gcol = lax.broadcasted_iota(jnp.int32, (G_GRAPHS, 1), 0)
        for q in range(QSUB):
            bt = bt_ref[q]
            onehot = (gcol == bt).astype(jnp.float32)
            hq = h[q * 128:(q + 1) * 128]
            pooled[...] += jnp.dot(onehot, hq,
                                   preferred_element_type=jnp.float32)
            counts[...] += jnp.sum(onehot, axis=1, keepdims=True)

        @pl.when(r == NRB - 1)
        def _():
            mean = pooled[...] / jnp.maximum(counts[...], 1.0)
            out_ref[...] = jnp.dot(mean, wl_ref[...],
                                   preferred_element_type=jnp.float32) \
                + bl_ref[...]

    return pl.pallas_call(
        body,
        grid=(NRB,),
        in_specs=[
            pl.BlockSpec((NC, RBLK, D), lambda r: (0, r, 0)),
            pl.BlockSpec((RBLK, D), lambda r: (r, 0)),
            pl.BlockSpec((NC, QSUB, 1, 128), lambda r: (0, r, 0, 0)),
            pl.BlockSpec((1, D), lambda r: (0, 0)),
            pl.BlockSpec((QSUB, 1, 128), lambda r: (r, 0, 0)),
            pl.BlockSpec((D, 128), lambda r: (0, 0)),
            pl.BlockSpec((1, 128), lambda r: (0, 0)),
        ],
        out_specs=pl.BlockSpec((G_GRAPHS, 128), lambda r: (0, 0)),
        out_shape=jax.ShapeDtypeStruct((G_GRAPHS, 128), jnp.float32),
        scratch_shapes=[
            pltpu.VMEM((G_GRAPHS, D), jnp.float32),
            pltpu.VMEM((G_GRAPHS, 1), jnp.float32),
        ],
    )(sp, gprev, degp, brow, batchp, wlp, blp)


def kernel(x, edge_index, batch, W1, b1, W2, b2, W3, b3, Wl, bl):
    e = edge_index.shape[1]
    pad = E_PAD - e
    src = jnp.concatenate([edge_index[0],
                           jnp.zeros((pad,), jnp.int32)]).reshape(NW, CPW,
                                                                  CHUNK)
    dst = jnp.concatenate([edge_index[1],
                           jnp.full((pad,), N_SC - 1, jnp.int32)]
                          ).reshape(NW, CPW, CHUNK)

    xp = jnp.pad(x, ((0, N_PAD - N_NODES), (0, 0)))
    batchp = jnp.pad(batch, (0, N_PAD - N_NODES),
                     constant_values=G_GRAPHS).reshape(NRB, QSUB, 1, 128)
    b1r = b1.reshape(1, D)
    b2r = b2.reshape(1, D)
    b3r = b3.reshape(1, D)
    wlp = jnp.pad(Wl, ((0, 0), (0, 128 - C_OUT)))
    blp = jnp.pad(bl, (0, 128 - C_OUT)).reshape(1, 128)

    idxall = jnp.stack([src.reshape(NW * CPW, CHUNK),
                        dst.reshape(NW * CPW, CHUNK)], axis=1)
    ncut = NS * CPW0
    zrow = jnp.zeros((2, 2, CHUNK), jnp.int32)
    idx0 = jnp.concatenate([idxall[:ncut], zrow])
    idx1 = jnp.concatenate([idxall[ncut:], zrow])

    deg = _sc_deg(dst)[:, :, 0, :ROWS_PT].reshape(NC, N_SC)
    degp = jnp.pad(deg, ((0, 0), (0, N_PAD - N_SC)),
                   constant_values=1.0).reshape(NC, NRB, QSUB, 1, 128)
    degp = degp.reshape(NC, NRB * QSUB, 1, 128)
    g1 = _tc_g1(degp, xp, W1)
    s1 = _sc_spmm(g1, idx0, idx1)
    g2 = _tc_layer(s1, g1, degp, b1r, W2)
    s2 = _sc_spmm(g2, idx0, idx1)
    g3 = _tc_layer(s2, g2, degp, b2r, W3)
    s3 = _sc_spmm(g3, idx0, idx1)
    batchf = batchp.reshape(NRB * QSUB, 1, 128)
    logits = _tc_final(s3, g3, degp, b3r, batchf, wlp, blp)
    return logits[:, :C_OUT]


# speedup vs baseline: 1.0753x; 1.0753x over previous
import functools

import jax
import jax.numpy as jnp
from jax import lax
from jax.experimental import pallas as pl
from jax.experimental.pallas import tpu as pltpu
from jax.experimental.pallas import tpu_sc as plsc

N_NODES = 10000
D = 128
G_GRAPHS = 128
C_OUT = 10

N_SC = 10112
N_PAD = 10240
RBLK = 512
NRB = N_PAD // RBLK
QSUB = RBLK // 128

NC, NS = 2, 16
NW = NC * NS
CHUNK = 128
CPW = 79
E_PAD = NW * CPW * CHUNK
ROWS_PT = N_SC // NS
CPW0, CPW1 = 149, 9

_MESH = plsc.VectorSubcoreMesh(core_axis_name="c", subcore_axis_name="s")


def _sc_deg(dsts):
    @functools.partial(
        pl.kernel,
        out_type=jax.ShapeDtypeStruct((NC, NS, 1, 640), jnp.float32),
        mesh=_MESH,
        scratch_types=[
            pltpu.VMEM((CPW, CHUNK), jnp.int32),
            pltpu.VMEM((CHUNK,), jnp.float32),
            pltpu.VMEM((640,), jnp.float32),
            pltpu.VMEM_SHARED((N_SC + 16,), jnp.float32),
        ],
    )
    def k(dst_hbm, deg_hbm, idx_v, ones_v, z_v, deg_sh):
        c = lax.axis_index("c")
        s = lax.axis_index("s")
        w = c * NS + s
        pltpu.sync_copy(dst_hbm.at[w], idx_v)

        @pl.loop(0, CHUNK // 16)
        def _(i):
            ones_v[pl.ds(i * 16, 16)] = jnp.full((16,), 1.0, jnp.float32)

        @pl.loop(0, 640 // 16)
        def _(i):
            z_v[pl.ds(i * 16, 16)] = jnp.zeros((16,), jnp.float32)

        pltpu.sync_copy(z_v.at[pl.ds(0, ROWS_PT)],
                        deg_sh.at[pl.ds(s * ROWS_PT, ROWS_PT)])
        plsc.subcore_barrier()

        @pl.loop(0, CPW)
        def _(j):
            pltpu.sync_copy(ones_v, deg_sh.at[idx_v.at[j]], add=True)

        plsc.subcore_barrier()
        pltpu.sync_copy(deg_sh.at[pl.ds(s * ROWS_PT, 640)], z_v)
        pltpu.sync_copy(z_v, deg_hbm.at[c, s, 0, pl.ds(0, 640)])

    return k(dsts)


def _sc_spmm(g, idx0, idx1):
    @functools.partial(
        pl.kernel,
        out_type=jax.ShapeDtypeStruct((NC, N_PAD, D), jnp.float32),
        mesh=_MESH,
        scratch_types=[
            pltpu.VMEM((2, CHUNK), jnp.int32),
            pltpu.VMEM((2, CHUNK), jnp.int32),
            pltpu.VMEM((2, CHUNK), jnp.int32),
            pltpu.VMEM((2, CHUNK), jnp.int32),
            pltpu.VMEM((CHUNK, D), jnp.float32),
            pltpu.VMEM((CHUNK, D), jnp.float32),
            pltpu.VMEM_SHARED((N_SC, D), jnp.float32),
            pltpu.SemaphoreType.DMA,
            pltpu.SemaphoreType.DMA,
            pltpu.SemaphoreType.DMA,
            pltpu.SemaphoreType.DMA,
            pltpu.SemaphoreType.DMA,
            pltpu.SemaphoreType.DMA,
        ],
    )
    def k(g_hbm, idx0_hbm, idx1_hbm, out_hbm,
          ixa, ixb, ixc, ixd, bufa, bufb, acc_sh,
          sga, sgb, sia, sib, sic, sid):
        c = lax.axis_index("c")
        s = lax.axis_index("s")

        @pl.loop(0, CHUNK)
        def _(i):
            for t in range(D // 16):
                bufa[i, pl.ds(t * 16, 16)] = jnp.zeros((16,), jnp.float32)

        base = s * ROWS_PT
        for off, nrows in ((0, 128), (128, 128), (256, 128), (384, 128),
                           (512, 120)):
            pltpu.sync_copy(bufa.at[pl.ds(0, nrows)],
                            acc_sh.at[pl.ds(base + off, nrows)])
        plsc.subcore_barrier()

        def pipeline(idx_hbm, cpw):
            jb = s * cpw
            pltpu.sync_copy(idx_hbm.at[jb], ixa)
            pltpu.async_copy(g_hbm.at[ixa.at[0]], bufa, sga)
            pltpu.async_copy(idx_hbm.at[jb + 1], ixb, sib)
            pltpu.async_copy(idx_hbm.at[jb + 2], ixc, sic)

            @pl.loop(0, (cpw - 1) // 4)
            def _(i):
                j = i * 4
                pltpu.make_async_copy(idx_hbm.at[jb + j + 1], ixb, sib).wait()
                pltpu.async_copy(g_hbm.at[ixb.at[0]], bufb, sgb)
                pltpu.make_async_copy(g_hbm.at[ixa.at[0]], bufa, sga).wait()
                pltpu.sync_copy(bufa, acc_sh.at[ixa.at[1]], add=True)
                pltpu.async_copy(idx_hbm.at[jb + j + 3], ixd, sid)
                pltpu.make_async_copy(idx_hbm.at[jb + j + 2], ixc, sic).wait()
                pltpu.async_copy(g_hbm.at[ixc.at[0]], bufa, sga)
                pltpu.make_async_copy(g_hbm.at[ixb.at[0]], bufb, sgb).wait()
                pltpu.sync_copy(bufb, acc_sh.at[ixb.at[1]], add=True)
                pltpu.async_copy(idx_hbm.at[jb + j + 4], ixa, sia)
                pltpu.make_async_copy(idx_hbm.at[jb + j + 3], ixd, sid).wait()
                pltpu.async_copy(g_hbm.at[ixd.at[0]], bufb, sgb)
                pltpu.make_async_copy(g_hbm.at[ixc.at[0]], bufa, sga).wait()
                pltpu.sync_copy(bufa, acc_sh.at[ixc.at[1]], add=True)
                pltpu.async_copy(idx_hbm.at[jb + j + 5], ixb, sib)
                pltpu.make_async_copy(idx_hbm.at[jb + j + 4], ixa, sia).wait()
                pltpu.async_copy(g_hbm.at[ixa.at[0]], bufa, sga)
                pltpu.make_async_copy(g_hbm.at[ixd.at[0]], bufb, sgb).wait()
                pltpu.sync_copy(bufb, acc_sh.at[ixd.at[1]], add=True)
                pltpu.async_copy(idx_hbm.at[jb + j + 6], ixc, sic)

            pltpu.make_async_copy(g_hbm.at[ixa.at[0]], bufa, sga).wait()
            pltpu.sync_copy(bufa, acc_sh.at[ixa.at[1]], add=True)
            pltpu.make_async_copy(idx_hbm.at[jb + cpw], ixb, sib).wait()
            pltpu.make_async_copy(idx_hbm.at[jb + cpw + 1], ixc, sic).wait()

        @pl.when(c == 0)
        def _():
            pipeline(idx0_hbm, CPW0)

        @pl.when(c == 1)
        def _():
            pipeline(idx1_hbm, CPW1)

        plsc.subcore_barrier()
        for off, nrows in ((0, 128), (128, 128), (256, 128), (384, 128),
                           (512, 120)):
            pltpu.sync_copy(acc_sh.at[pl.ds(base + off, nrows)],
                            bufa.at[pl.ds(0, nrows)])
            pltpu.sync_copy(bufa.at[pl.ds(0, nrows)],
                            out_hbm.at[c, pl.ds(base + off, nrows)])

    return k(g, idx0, idx1)


def _dinv_block(deg_ref):
    ii = lax.broadcasted_iota(jnp.int32, (128, 128), 0)
    jj = lax.broadcasted_iota(jnp.int32, (128, 128), 1)
    ones = jnp.ones((128, 128), jnp.float32)
    pieces = []
    for q in range(QSUB):
        d = deg_ref[0, q] + deg_ref[1, q] + 1.0
        row = lax.rsqrt(d)
        diag = jnp.where(ii == jj, jnp.broadcast_to(row, (128, 128)), 0.0)
        pieces.append(jnp.dot(diag, ones,
                              preferred_element_type=jnp.float32))
    return jnp.concatenate(pieces, axis=0)


def _tc_g1(degp, x, w1):
    def body(deg_ref, x_ref, w_ref, g_ref):
        dinvb = _dinv_block(deg_ref)
        g_ref[...] = dinvb * jnp.dot(x_ref[...], w_ref[...],
                                     preferred_element_type=jnp.float32)

    return pl.pallas_call(
        body,
        grid=(NRB,),
        in_specs=[
            pl.BlockSpec((NC, QSUB, 1, 128), lambda r: (0, r, 0, 0)),
            pl.BlockSpec((RBLK, D), lambda r: (r, 0)),
            pl.BlockSpec((D, D), lambda r: (0, 0)),
        ],
        out_specs=pl.BlockSpec((RBLK, D), lambda r: (r, 0)),
        out_shape=jax.ShapeDtypeStruct((N_PAD, D), jnp.float32),
    )(degp, x, w1)


def _tc_layer(sp, gprev, degp, brow, w):
    def body(sp_ref, g_ref, deg_ref, b_ref, w_ref, out_ref):
        dinvb = _dinv_block(deg_ref)
        ssum = sp_ref[0] + sp_ref[1]
        a = jnp.maximum(dinvb * (ssum + g_ref[...]) + b_ref[...], 0.0)
        out_ref[...] = dinvb * jnp.dot(a, w_ref[...],
                                       preferred_element_type=jnp.float32)

    return pl.pallas_call(
        body,
        grid=(NRB,),
        in_specs=[
            pl.BlockSpec((NC, RBLK, D), lambda r: (0, r, 0)),
            pl.BlockSpec((RBLK, D), lambda r: (r, 0)),
            pl.BlockSpec((NC, QSUB, 1, 128), lambda r: (0, r, 0, 0)),
            pl.BlockSpec((1, D), lambda r: (0, 0)),
            pl.BlockSpec((D, D), lambda r: (0, 0)),
        ],
        out_specs=pl.BlockSpec((RBLK, D), lambda r: (r, 0)),
        out_shape=jax.ShapeDtypeStruct((N_PAD, D), jnp.float32),
    )(sp, gprev, degp, brow, w)


def _tc_final(sp, gprev, degp, brow, batchp, wlp, blp):
    def body(sp_ref, g_ref, deg_ref, b_ref, bt_ref, wl_ref, bl_ref, out_ref,
             pooled, counts):
        r = pl.program_id(0)

        @pl.when(r == 0)
        def _():
            pooled[...] = jnp.zeros_like(pooled)
            counts[...] = jnp.zeros_like(counts)

        dinvb = _dinv_block(deg_ref)
        h = dinvb * (sp_ref[0] + sp_ref[1] + g_ref[...]) + b_ref[...]
        rid = r * RBLK + lax.broadcasted_iota(jnp.int32, (RBLK, 1), 0)
        h = jnp.where(rid < N_NODES, h, 0.0)
        gcol = lax.broadcasted_iota(jnp.int32, (G_GRAPHS, 1), 0)
        for q in range(QSUB):
            bt = bt_ref[q]
            onehot = (gcol == bt).astype(jnp.float32)
            hq = h[q * 128:(q + 1) * 128]
            pooled[...] += jnp.dot(onehot, hq,
                                   preferred_element_type=jnp.float32)
            counts[...] += jnp.sum(onehot, axis=1, keepdims=True)

        @pl.when(r == NRB - 1)
        def _():
            mean = pooled[...] / jnp.maximum(counts[...], 1.0)
            out_ref[...] = jnp.dot(mean, wl_ref[...],
                                   preferred_element_type=jnp.float32) \
                + bl_ref[...]

    return pl.pallas_call(
        body,
        grid=(NRB,),
        in_specs=[
            pl.BlockSpec((NC, RBLK, D), lambda r: (0, r, 0)),
            pl.BlockSpec((RBLK, D), lambda r: (r, 0)),
            pl.BlockSpec((NC, QSUB, 1, 128), lambda r: (0, r, 0, 0)),
            pl.BlockSpec((1, D), lambda r: (0, 0)),
            pl.BlockSpec((QSUB, 1, 128), lambda r: (r, 0, 0)),
            pl.BlockSpec((D, 128), lambda r: (0, 0)),
            pl.BlockSpec((1, 128), lambda r: (0, 0)),
        ],
        out_specs=pl.BlockSpec((G_GRAPHS, 128), lambda r: (0, 0)),
        out_shape=jax.ShapeDtypeStruct((G_GRAPHS, 128), jnp.float32),
        scratch_shapes=[
            pltpu.VMEM((G_GRAPHS, D), jnp.float32),
            pltpu.VMEM((G_GRAPHS, 1), jnp.float32),
        ],
    )(sp, gprev, degp, brow, batchp, wlp, blp)


def kernel(x, edge_index, batch, W1, b1, W2, b2, W3, b3, Wl, bl):
    e = edge_index.shape[1]
    pad = E_PAD - e
    src = jnp.concatenate([edge_index[0],
                           jnp.zeros((pad,), jnp.int32)]).reshape(NW, CPW,
                                                                  CHUNK)
    dst = jnp.concatenate([edge_index[1],
                           jnp.full((pad,), N_SC - 1, jnp.int32)]
                          ).reshape(NW, CPW, CHUNK)

    xp = jnp.pad(x, ((0, N_PAD - N_NODES), (0, 0)))
    batchp = jnp.pad(batch, (0, N_PAD - N_NODES),
                     constant_values=G_GRAPHS).reshape(NRB, QSUB, 1, 128)
    b1r = b1.reshape(1, D)
    b2r = b2.reshape(1, D)
    b3r = b3.reshape(1, D)
    wlp = jnp.pad(Wl, ((0, 0), (0, 128 - C_OUT)))
    blp = jnp.pad(bl, (0, 128 - C_OUT)).reshape(1, 128)

    idxall = jnp.stack([src.reshape(NW * CPW, CHUNK),
                        dst.reshape(NW * CPW, CHUNK)], axis=1)
    ncut = NS * CPW0
    zrow = jnp.zeros((2, 2, CHUNK), jnp.int32)
    idx0 = jnp.concatenate([idxall[:ncut], zrow])
    idx1 = jnp.concatenate([idxall[ncut:], zrow])

    deg = _sc_deg(dst)[:, :, 0, :ROWS_PT].reshape(NC, N_SC)
    degp = jnp.pad(deg, ((0, 0), (0, N_PAD - N_SC)),
                   constant_values=1.0).reshape(NC, NRB, QSUB, 1, 128)
    degp = degp.reshape(NC, NRB * QSUB, 1, 128)
    g1 = _tc_g1(degp, xp, W1)
    s1 = _sc_spmm(g1, idx0, idx1)
    g2 = _tc_layer(s1, g1, degp, b1r, W2)
    s2 = _sc_spmm(g2, idx0, idx1)
    g3 = _tc_layer(s2, g2, degp, b2r, W3)
    s3 = _sc_spmm(g3, idx0, idx1)
    batchf = batchp.reshape(NRB * QSUB, 1, 128)
    logits = _tc_final(s3, g3, degp, b3r, batchf, wlp, blp)
    return logits[:, :C_OUT]
